# baseline (speedup 1.0000x reference)
import functools
import jax
import jax.numpy as jnp
from jax import lax
from jax.experimental import pallas as pl
from jax.experimental.pallas import tpu as pltpu, tpu_sc as plsc

BATCH = 1024
L = 200
N = 16
D = 8
EMB = N * D
T = BATCH * L

TPG = 8
GROWS = T // TPG

NC = 2
NS = 16
NW = NC * NS
RPW = GROWS // NW

CR = L
NCHUNK = RPW // CR


def _sc_gather_kernel(w_hbm, x_hbm, emb_hbm, idx_v, rows_v, pack_v, sem):
    wid = lax.axis_index("s") * NC + lax.axis_index("c")
    row0w = wid * RPW

    def chunk_body(ci, carry):
        row0 = row0w + ci * CR
        brow = row0 // L
        for j in range(TPG):
            pltpu.sync_copy(x_hbm.at[pl.ds(128 * j + brow, 1)],
                            idx_v.at[pl.ds(j, 1)])
        copies = []
        for j in range(TPG):
            copies.append(pltpu.async_copy(
                w_hbm.at[idx_v.at[j, pl.ds(0, 128)]],
                rows_v.at[pl.ds(L * j, 128)], sem))
            copies.append(pltpu.async_copy(
                w_hbm.at[idx_v.at[j, pl.ds(128, L - 128)]],
                rows_v.at[pl.ds(L * j + 128, L - 128)], sem))
        for cp in copies:
            cp.wait()

        def row_body(r, rc):
            for j in range(TPG):
                pack_v[r, pl.ds(16 * j, 16)] = rows_v[L * j + r]
            return rc

        lax.fori_loop(0, CR, row_body, 0, unroll=4)

        pltpu.sync_copy(pack_v, emb_hbm.at[pl.ds(row0, CR)])
        return carry

    lax.fori_loop(0, NCHUNK, chunk_body, 0)


def _tc_expand_kernel(emb_ref, g_ref, out_ref):
    j = pl.program_id(1)
    xb = emb_ref[...]
    out_ref[...] = jnp.dot(
        xb, g_ref[j], preferred_element_type=jnp.float32
    ).reshape(out_ref.shape)


BR = 1600
NI = GROWS // BR
BB = BR // L


@jax.jit
def _run(w, x, g):
    mesh = plsc.VectorSubcoreMesh(core_axis_name="c", subcore_axis_name="s")
    gather = functools.partial(
        pl.kernel,
        out_type=jax.ShapeDtypeStruct((GROWS, EMB), jnp.float32),
        mesh=mesh,
        scratch_types=[
            pltpu.VMEM((TPG, L), jnp.int32),
            pltpu.VMEM((TPG * L, N), jnp.float32),
            pltpu.VMEM((CR, EMB), jnp.float32),
            pltpu.SemaphoreType.DMA,
        ],
        compiler_params=pltpu.CompilerParams(use_tc_tiling_on_sc=False),
    )(_sc_gather_kernel)
    emb2 = gather(w, x)

    return pl.pallas_call(
        _tc_expand_kernel,
        grid=(NI, TPG),
        in_specs=[
            pl.BlockSpec((BR, EMB), lambda i, j: (i, 0)),
            pl.BlockSpec((TPG, EMB, EMB), lambda i, j: (0, 0, 0)),
        ],
        out_specs=pl.BlockSpec((BB, L, EMB), lambda i, j: (j * NI + i, 0, 0)),
        out_shape=jax.ShapeDtypeStruct((BATCH, L, EMB), jnp.float32),
    )(emb2, g)


def kernel(x, W, B, p):
    p = p.astype(jnp.int32)
    perm_idx = p // D
    scale = B[0, p % D].astype(jnp.float32)
    jj = jnp.arange(TPG, dtype=jnp.int32)[:, None]
    kk = jnp.arange(EMB, dtype=jnp.int32)[None, :]
    g = jnp.zeros((TPG, EMB, EMB), jnp.float32)
    g = g.at[jnp.broadcast_to(jj, (TPG, EMB)),
             16 * jj + perm_idx[None, :],
             jnp.broadcast_to(kk, (TPG, EMB))].set(
        jnp.broadcast_to(scale[None, :], (TPG, EMB)))
    return _run(W, x.astype(jnp.int32), g)


# speedup vs baseline: 1.0493x; 1.0493x over previous
import functools
import jax
import jax.numpy as jnp
from jax import lax
from jax.experimental import pallas as pl
from jax.experimental.pallas import tpu as pltpu, tpu_sc as plsc

BATCH = 1024
L = 200
N = 16
D = 8
EMB = N * D
T = BATCH * L

NC = 2
NS = 16
NW = NC * NS
TPW = T // NW

C = 640
K = C // 128
NCHUNK = TPW // C


def _sc_expand_kernel(w_hbm, x_hbm, perm_hbm, scale_hbm, out_hbm,
                      idx_v, rows_v, out_v, perm_v, scale_v, sem):
    wid = lax.axis_index("s") * NC + lax.axis_index("c")

    pltpu.sync_copy(perm_hbm, perm_v)
    pltpu.sync_copy(scale_hbm, scale_v)
    perm_regs = [perm_v[pl.ds(16 * g, 16)] for g in range(D)]
    scale_regs = [scale_v[pl.ds(16 * g, 16)] for g in range(D)]

    tokw0 = wid * TPW

    def chunk_body(ci, carry):
        tok0 = tokw0 + ci * C
        pltpu.sync_copy(x_hbm.at[pl.ds(tok0, C)], idx_v)
        copies = [
            pltpu.async_copy(w_hbm.at[idx_v.at[pl.ds(j * 128, 128)]],
                             rows_v.at[pl.ds(j * 128, 128)], sem)
            for j in range(K)
        ]
        for c in copies:
            c.wait()

        dnums = lax.GatherDimensionNumbers(
            offset_dims=(), collapsed_slice_dims=(0,), start_index_map=(0,))

        def tok_body(t, tc):
            emb = rows_v[t]
            for g in range(D):
                vals = lax.gather(
                    emb, perm_regs[g][:, None], dnums, slice_sizes=(1,),
                    mode=lax.GatherScatterMode.PROMISE_IN_BOUNDS)
                out_v[t, pl.ds(16 * g, 16)] = vals * scale_regs[g]
            return tc

        lax.fori_loop(0, C, tok_body, 0, unroll=2)
        pltpu.sync_copy(out_v, out_hbm.at[pl.ds(tok0, C)])
        return carry

    lax.fori_loop(0, NCHUNK, chunk_body, 0)


@jax.jit
def _run(w, x1, perm_idx, scale):
    mesh = plsc.VectorSubcoreMesh(core_axis_name="c", subcore_axis_name="s")
    kfn = functools.partial(
        pl.kernel,
        out_type=jax.ShapeDtypeStruct((T, EMB), jnp.float32),
        mesh=mesh,
        scratch_types=[
            pltpu.VMEM((C,), jnp.int32),
            pltpu.VMEM((C, N), jnp.float32),
            pltpu.VMEM((C, EMB), jnp.float32),
            pltpu.VMEM((EMB,), jnp.int32),
            pltpu.VMEM((EMB,), jnp.float32),
            pltpu.SemaphoreType.DMA,
        ],
        compiler_params=pltpu.CompilerParams(use_tc_tiling_on_sc=False),
    )(_sc_expand_kernel)
    return kfn(w, x1, perm_idx, scale)


def kernel(x, W, B, p):
    p = p.astype(jnp.int32)
    perm_idx = p // D
    scale = B[0, p % D].astype(jnp.float32)
    x1 = x.astype(jnp.int32).reshape(T)
    out = _run(W, x1, perm_idx, scale)
    return out.reshape(BATCH, L, EMB)
